# baseline (speedup 1.0000x reference)
import functools

import jax
import jax.numpy as jnp
from jax import lax
from jax.experimental import pallas as pl
from jax.experimental.pallas import tpu as pltpu
from jax.experimental.pallas import tpu_sc as plsc

M, D1, D2 = 100000, 64, 4
NCOL = D1 * D2
NOUT = M * NCOL
B = 16384
NUPD = B * NCOL

NC, NS = 2, 16
NWIN = 20
E = NOUT // NWIN
WPC = NWIN // NC
DUMMY = 32768
SEG = E // NS
STG = SEG // 10
UPT = NUPD // NS
CHUNK = 8192
NCH = UPT // CHUNK
VPC = CHUNK // 16


def _sc_scatter_add(inp_flat, idx_flat, src_flat):
    mesh = plsc.VectorSubcoreMesh(core_axis_name="c", subcore_axis_name="s")

    @functools.partial(
        pl.kernel,
        out_type=jax.ShapeDtypeStruct((NOUT,), jnp.float32),
        mesh=mesh,
        scratch_types=[
            pltpu.VMEM_SHARED((E + DUMMY,), jnp.float32),
            pltpu.VMEM((CHUNK,), jnp.int32),
            pltpu.VMEM((CHUNK,), jnp.float32),
            pltpu.VMEM((CHUNK,), jnp.int32),
            pltpu.VMEM((STG,), jnp.float32),
            pltpu.SemaphoreType.DMA,
        ],
    )
    def k(in_hbm, idx_hbm, src_hbm, out_hbm, win, idxv, srcv, dl, stg, sem):
        c = lax.axis_index("c")
        s = lax.axis_index("s")
        iota = lax.iota(jnp.int32, 16)
        for p in range(WPC):
            w0 = (c * WPC + p) * E
            @pl.loop(0, SEG // STG)
            def _init(q):
                off = s * SEG + q * STG
                pltpu.sync_copy(in_hbm.at[pl.ds(w0 + off, STG)], stg)
                pltpu.sync_copy(stg, win.at[pl.ds(off, STG)])

            plsc.subcore_barrier()

            @pl.loop(0, NCH)
            def _chunk(kk):
                base = s * UPT + kk * CHUNK
                pltpu.sync_copy(idx_hbm.at[pl.ds(base, CHUNK)], idxv)
                pltpu.sync_copy(src_hbm.at[pl.ds(base, CHUNK)], srcv)

                @pl.loop(0, VPC)
                def _vec(v):
                    iv = idxv[pl.ds(v * 16, 16)]
                    colb = (v % 16) * 16
                    rel = (iv << 8) + ((colb - w0) + iota)
                    inw = plsc.bitcast(rel, jnp.uint32) < jnp.uint32(E)
                    dmy = E + (rel & (DUMMY - 1))
                    dl[pl.ds(v * 16, 16)] = jnp.where(inw, rel, dmy)

                pltpu.sync_copy(srcv, win.at[dl], add=True)

            plsc.subcore_barrier()

            @pl.loop(0, SEG // STG)
            def _wb(q):
                off = s * SEG + q * STG
                pltpu.sync_copy(win.at[pl.ds(off, STG)], stg)
                pltpu.sync_copy(stg, out_hbm.at[pl.ds(w0 + off, STG)])

    return k(inp_flat, idx_flat, src_flat)


def kernel(input, index, src):
    out_flat = _sc_scatter_add(
        input.reshape(NOUT), index.reshape(NUPD), src.reshape(NUPD))
    return out_flat.reshape(M, D1, D2)


# speedup vs baseline: 37.7715x; 37.7715x over previous
import functools

import jax
import jax.numpy as jnp
from jax import lax
from jax.experimental import pallas as pl
from jax.experimental.pallas import tpu as pltpu
from jax.experimental.pallas import tpu_sc as plsc

M, D1, D2 = 100000, 64, 4
B = 16384

NC, NS = 2, 16
SPC = D1 // NC
SLAB = D2 * M
MAIN = 99968
TAILM = M - MAIN
PIECE = 6272
TPIECE = MAIN - 15 * PIECE
UPT = B // NS
VPT = UPT // 16


def _sc_scatter_add(in3, idx3, src3):
    mesh = plsc.VectorSubcoreMesh(core_axis_name="c", subcore_axis_name="s")

    @functools.partial(
        pl.kernel,
        out_type=(
            jax.ShapeDtypeStruct((D1, D2, M), jnp.float32),
            jax.ShapeDtypeStruct((D1 * 128, ), jnp.float32),
        ),
        mesh=mesh,
        scratch_types=[
            pltpu.VMEM_SHARED((SLAB,), jnp.float32),
            pltpu.VMEM((D2, PIECE), jnp.float32),
            pltpu.VMEM((D2 * PIECE,), jnp.float32),
            pltpu.VMEM((D2, UPT), jnp.int32),
            pltpu.VMEM((D2, UPT), jnp.float32),
            pltpu.VMEM((D2 * UPT,), jnp.int32),
            pltpu.VMEM((D2 * UPT,), jnp.float32),
            pltpu.VMEM((128,), jnp.float32),
        ],
    )
    def k(in_hbm, idx_hbm, src_hbm, out_hbm, tout_hbm, win, stg, flat, idxv,
          srcv, dl, srcfl, tbuf):
        c = lax.axis_index("c")
        s = lax.axis_index("s")
        m0 = s * PIECE
        i0 = s * UPT

        def flatten(n, fwd):
            for kx in range(D2):
                @pl.loop(0, n // 16, step=4)
                def _f(v):
                    for u in range(4):
                        o = (v + u) * 16
                        if fwd:
                            flat[pl.ds(kx * n + o, 16)] = stg[kx, pl.ds(o, 16)]
                        else:
                            stg[kx, pl.ds(o, 16)] = flat[pl.ds(kx * n + o, 16)]

        for t in range(128 // 16):
            tbuf[pl.ds(t * 16, 16)] = jnp.zeros((16,), jnp.float32)

        @pl.loop(0, SPC)
        def _slab(jj):
            j = c * SPC + jj

            @pl.when(s < NS - 1)
            def _init_full():
                pltpu.sync_copy(in_hbm.at[j, :, pl.ds(m0, PIECE)], stg)
                flatten(PIECE, True)
                for kx in range(D2):
                    pltpu.sync_copy(flat.at[pl.ds(kx * PIECE, PIECE)],
                                    win.at[pl.ds(kx * M + m0, PIECE)])

            @pl.when(s == NS - 1)
            def _init_tail():
                pltpu.sync_copy(in_hbm.at[j, :, pl.ds(m0, TPIECE)],
                                stg.at[:, pl.ds(0, TPIECE)])
                flatten(TPIECE, True)
                for kx in range(D2):
                    pltpu.sync_copy(flat.at[pl.ds(kx * TPIECE, TPIECE)],
                                    win.at[pl.ds(kx * M + m0, TPIECE)])
                    pltpu.sync_copy(tbuf.at[pl.ds(0, TAILM)],
                                    win.at[pl.ds(kx * M + MAIN, TAILM)])

            plsc.subcore_barrier()

            pltpu.sync_copy(idx_hbm.at[j, :, pl.ds(i0, UPT)], idxv)
            pltpu.sync_copy(src_hbm.at[j, :, pl.ds(i0, UPT)], srcv)
            for kx in range(D2):
                @pl.loop(0, VPT, step=4)
                def _vec(v):
                    for u in range(4):
                        o = (v + u) * 16
                        dl[pl.ds(kx * UPT + o, 16)] = (
                            idxv[kx, pl.ds(o, 16)] + kx * M)
                        srcfl[pl.ds(kx * UPT + o, 16)] = srcv[kx, pl.ds(o, 16)]

            pltpu.sync_copy(srcfl, win.at[dl], add=True)

            plsc.subcore_barrier()

            @pl.when(s < NS - 1)
            def _wb_full():
                for kx in range(D2):
                    pltpu.sync_copy(win.at[pl.ds(kx * M + m0, PIECE)],
                                    flat.at[pl.ds(kx * PIECE, PIECE)])
                flatten(PIECE, False)
                pltpu.sync_copy(stg, out_hbm.at[j, :, pl.ds(m0, PIECE)])

            @pl.when(s == NS - 1)
            def _wb_tail():
                for kx in range(D2):
                    pltpu.sync_copy(win.at[pl.ds(kx * M + m0, TPIECE)],
                                    flat.at[pl.ds(kx * TPIECE, TPIECE)])
                flatten(TPIECE, False)
                pltpu.sync_copy(stg.at[:, pl.ds(0, TPIECE)],
                                out_hbm.at[j, :, pl.ds(m0, TPIECE)])
                for kx in range(D2):
                    pltpu.sync_copy(win.at[pl.ds(kx * M + MAIN, TAILM)],
                                    tbuf.at[pl.ds(kx * TAILM, TAILM)])
                pltpu.sync_copy(tbuf, tout_hbm.at[pl.ds(j * 128, 128)])
                for t in range(128 // 16):
                    tbuf[pl.ds(t * 16, 16)] = jnp.zeros((16,), jnp.float32)

    return k(in3, idx3, src3)


def kernel(input, index, src):
    out3, tout = _sc_scatter_add(
        jnp.transpose(input, (1, 2, 0)),
        jnp.transpose(index, (1, 2, 0)),
        jnp.transpose(src, (1, 2, 0)),
    )
    out = jnp.transpose(out3, (2, 0, 1))
    tail = jnp.transpose(tout.reshape(D1, D2, TAILM), (2, 0, 1))
    tail = tail + lax.slice(input, (MAIN, 0, 0), (M, D1, D2))
    return lax.dynamic_update_slice(out, tail, (MAIN, 0, 0))


# speedup vs baseline: 58.3606x; 1.5451x over previous
import functools

import jax
import jax.numpy as jnp
from jax import lax
from jax.experimental import pallas as pl
from jax.experimental.pallas import tpu as pltpu
from jax.experimental.pallas import tpu_sc as plsc

M, D1, D2 = 100000, 64, 4
B = 16384

NC, NS = 2, 16
SPC = D1 // NC
SLAB = D2 * M
MAIN = 99968
TAILM = M - MAIN
PIECE = 6272
UPT = B // NS
VPT = UPT // 16


def _sc_scatter_add(in3, idx3, src3):
    mesh = plsc.VectorSubcoreMesh(core_axis_name="c", subcore_axis_name="s")

    @functools.partial(
        pl.kernel,
        out_type=(
            jax.ShapeDtypeStruct((D1, D2, M), jnp.float32),
            jax.ShapeDtypeStruct((D1 * 128, ), jnp.float32),
        ),
        mesh=mesh,
        scratch_types=[
            pltpu.VMEM_SHARED((SLAB,), jnp.float32),
            pltpu.VMEM((D2, PIECE), jnp.float32),
            pltpu.VMEM((D2 * PIECE,), jnp.float32),
            pltpu.VMEM((D2, UPT), jnp.int32),
            pltpu.VMEM((D2, UPT), jnp.float32),
            pltpu.VMEM((D2 * UPT,), jnp.int32),
            pltpu.VMEM((D2 * UPT,), jnp.float32),
            pltpu.VMEM((128,), jnp.float32),
            pltpu.VMEM((PIECE,), jnp.float32),
            pltpu.SemaphoreType.DMA,
            pltpu.SemaphoreType.DMA,
            pltpu.SemaphoreType.DMA,
            pltpu.SemaphoreType.DMA,
        ],
    )
    def k(in_hbm, idx_hbm, src_hbm, out_hbm, tout_hbm, win, stg, flat, idxv,
          srcv, dl, srcfl, tbuf, zflat, semi, semu, semz, semw):
        c = lax.axis_index("c")
        s = lax.axis_index("s")
        m0 = jnp.minimum(s * PIECE, MAIN - PIECE)
        i0 = s * UPT
        j0 = c * SPC
        last = NS - 1

        @pl.loop(0, PIECE // 16, step=4)
        def _z(v):
            for u in range(4):
                zflat[pl.ds((v + u) * 16, 16)] = jnp.zeros((16,), jnp.float32)

        def issue_updates(j):
            pltpu.async_copy(idx_hbm.at[j, :, pl.ds(i0, UPT)], idxv, semu)
            pltpu.async_copy(src_hbm.at[j, :, pl.ds(i0, UPT)], srcv, semu)

        def wait_updates(j):
            pltpu.make_async_copy(
                idx_hbm.at[j, :, pl.ds(i0, UPT)], idxv, semu).wait()
            pltpu.make_async_copy(
                src_hbm.at[j, :, pl.ds(i0, UPT)], srcv, semu).wait()

        def issue_input(j):
            pltpu.async_copy(in_hbm.at[j, :, pl.ds(m0, PIECE)], stg, semi)

        def wait_input(j):
            pltpu.make_async_copy(
                in_hbm.at[j, :, pl.ds(m0, PIECE)], stg, semi).wait()

        def issue_zero():
            for kx in range(D2):
                pltpu.async_copy(
                    zflat, win.at[pl.ds(kx * M + m0, PIECE)], semz)

            @pl.when(s == last)
            def _tz():
                for kx in range(D2):
                    pltpu.async_copy(zflat.at[pl.ds(0, TAILM)],
                                     win.at[pl.ds(kx * M + MAIN, TAILM)],
                                     semz)

        def wait_zero():
            for kx in range(D2):
                pltpu.make_async_copy(
                    zflat, win.at[pl.ds(kx * M + m0, PIECE)], semz).wait()

            @pl.when(s == last)
            def _tzw():
                for kx in range(D2):
                    pltpu.make_async_copy(
                        zflat.at[pl.ds(0, TAILM)],
                        win.at[pl.ds(kx * M + MAIN, TAILM)], semz).wait()

        issue_updates(j0)
        issue_input(j0)
        issue_zero()

        @pl.loop(0, SPC)
        def _slab(jj):
            j = j0 + jj

            wait_updates(j)
            for kx in range(D2):
                @pl.loop(0, VPT, step=4)
                def _vec(v):
                    for u in range(4):
                        o = (v + u) * 16
                        dl[pl.ds(kx * UPT + o, 16)] = (
                            idxv[kx, pl.ds(o, 16)] + kx * M)
                        srcfl[pl.ds(kx * UPT + o, 16)] = srcv[kx, pl.ds(o, 16)]

            @pl.when(jj < SPC - 1)
            def _pfu():
                issue_updates(j + 1)

            wait_zero()
            plsc.subcore_barrier()

            pltpu.sync_copy(srcfl, win.at[dl], add=True)

            plsc.subcore_barrier()

            for kx in range(D2):
                pltpu.async_copy(win.at[pl.ds(kx * M + m0, PIECE)],
                                 flat.at[pl.ds(kx * PIECE, PIECE)], semw)
            for kx in range(D2):
                pltpu.make_async_copy(
                    win.at[pl.ds(kx * M + m0, PIECE)],
                    flat.at[pl.ds(kx * PIECE, PIECE)], semw).wait()

            @pl.when(s == last)
            def _tout():
                for kx in range(D2):
                    pltpu.sync_copy(win.at[pl.ds(kx * M + MAIN, TAILM)],
                                    tbuf.at[pl.ds(kx * TAILM, TAILM)])
                pltpu.sync_copy(tbuf, tout_hbm.at[pl.ds(j * 128, 128)])

            @pl.when(jj < SPC - 1)
            def _rz():
                issue_zero()

            wait_input(j)
            for kx in range(D2):
                @pl.loop(0, PIECE // 16, step=4)
                def _add(v):
                    for u in range(4):
                        o = (v + u) * 16
                        stg[kx, pl.ds(o, 16)] = (
                            stg[kx, pl.ds(o, 16)]
                            + flat[pl.ds(kx * PIECE + o, 16)])

            pltpu.sync_copy(stg, out_hbm.at[j, :, pl.ds(m0, PIECE)])

            @pl.when(jj < SPC - 1)
            def _pfi():
                issue_input(j + 1)

    return k(in3, idx3, src3)


def kernel(input, index, src):
    out3, tout = _sc_scatter_add(
        jnp.transpose(input, (1, 2, 0)),
        jnp.transpose(index, (1, 2, 0)),
        jnp.transpose(src, (1, 2, 0)),
    )
    out = jnp.transpose(out3, (2, 0, 1))
    tail = jnp.transpose(tout.reshape(D1, D2, TAILM), (2, 0, 1))
    tail = tail + lax.slice(input, (MAIN, 0, 0), (M, D1, D2))
    return lax.dynamic_update_slice(out, tail, (MAIN, 0, 0))


# speedup vs baseline: 64.1055x; 1.0984x over previous
import functools

import jax
import jax.numpy as jnp
from jax import lax
from jax.experimental import pallas as pl
from jax.experimental.pallas import tpu as pltpu
from jax.experimental.pallas import tpu_sc as plsc

M, D1, D2 = 100000, 64, 4
B = 16384

NC, NS = 2, 16
SPC = D1 // NC
SLAB = D2 * M
MAIN = 99968
TAILM = M - MAIN
PIECE = 6272
UPT = B // NS
VPT = UPT // 16


def _sc_scatter_add(in3, idx3, src3):
    mesh = plsc.VectorSubcoreMesh(core_axis_name="c", subcore_axis_name="s")

    @functools.partial(
        pl.kernel,
        out_type=(
            jax.ShapeDtypeStruct((D1, D2, M), jnp.float32),
            jax.ShapeDtypeStruct((D1 * 128, ), jnp.float32),
        ),
        mesh=mesh,
        scratch_types=[
            pltpu.VMEM_SHARED((SLAB,), jnp.float32),
            pltpu.VMEM((D2, PIECE), jnp.float32),
            pltpu.VMEM((D2 * PIECE,), jnp.float32),
            pltpu.VMEM((D2, UPT), jnp.int32),
            pltpu.VMEM((D2, UPT), jnp.float32),
            pltpu.VMEM((D2 * UPT,), jnp.int32),
            pltpu.VMEM((D2 * UPT,), jnp.float32),
            pltpu.VMEM((128,), jnp.float32),
            pltpu.VMEM((PIECE,), jnp.float32),
            pltpu.VMEM((D2, PIECE), jnp.float32),
            pltpu.SemaphoreType.DMA,
            pltpu.SemaphoreType.DMA,
            pltpu.SemaphoreType.DMA,
            pltpu.SemaphoreType.DMA,
            pltpu.SemaphoreType.DMA,
        ],
    )
    def k(in_hbm, idx_hbm, src_hbm, out_hbm, tout_hbm, win, stg, flat, idxv,
          srcv, dl, srcfl, tbuf, zflat, stgo, semi, semu, semz, semw, semo):
        c = lax.axis_index("c")
        s = lax.axis_index("s")
        m0 = jnp.minimum(s * PIECE, MAIN - PIECE)
        i0 = s * UPT
        j0 = c * SPC
        last = NS - 1

        @pl.loop(0, PIECE // 16, step=4)
        def _z(v):
            for u in range(4):
                zflat[pl.ds((v + u) * 16, 16)] = jnp.zeros((16,), jnp.float32)

        def issue_updates(j):
            pltpu.async_copy(idx_hbm.at[j, :, pl.ds(i0, UPT)], idxv, semu)
            pltpu.async_copy(src_hbm.at[j, :, pl.ds(i0, UPT)], srcv, semu)

        def wait_updates(j):
            pltpu.make_async_copy(
                idx_hbm.at[j, :, pl.ds(i0, UPT)], idxv, semu).wait()
            pltpu.make_async_copy(
                src_hbm.at[j, :, pl.ds(i0, UPT)], srcv, semu).wait()

        def issue_input(j):
            pltpu.async_copy(in_hbm.at[j, :, pl.ds(m0, PIECE)], stg, semi)

        def wait_input(j):
            pltpu.make_async_copy(
                in_hbm.at[j, :, pl.ds(m0, PIECE)], stg, semi).wait()

        def issue_zero():
            for kx in range(D2):
                pltpu.async_copy(
                    zflat, win.at[pl.ds(kx * M + m0, PIECE)], semz)

            @pl.when(s == last)
            def _tz():
                for kx in range(D2):
                    pltpu.async_copy(zflat.at[pl.ds(0, TAILM)],
                                     win.at[pl.ds(kx * M + MAIN, TAILM)],
                                     semz)

        def wait_zero():
            for kx in range(D2):
                pltpu.make_async_copy(
                    zflat, win.at[pl.ds(kx * M + m0, PIECE)], semz).wait()

            @pl.when(s == last)
            def _tzw():
                for kx in range(D2):
                    pltpu.make_async_copy(
                        zflat.at[pl.ds(0, TAILM)],
                        win.at[pl.ds(kx * M + MAIN, TAILM)], semz).wait()

        issue_updates(j0)
        issue_input(j0)
        issue_zero()

        @pl.loop(0, SPC)
        def _slab(jj):
            j = j0 + jj

            wait_updates(j)
            for kx in range(D2):
                @pl.loop(0, VPT, step=4)
                def _vec(v):
                    for u in range(4):
                        o = (v + u) * 16
                        dl[pl.ds(kx * UPT + o, 16)] = (
                            idxv[kx, pl.ds(o, 16)] + kx * M)
                        srcfl[pl.ds(kx * UPT + o, 16)] = srcv[kx, pl.ds(o, 16)]

            @pl.when(jj < SPC - 1)
            def _pfu():
                issue_updates(j + 1)

            wait_zero()
            plsc.subcore_barrier()

            pltpu.sync_copy(srcfl, win.at[dl], add=True)

            plsc.subcore_barrier()

            for kx in range(D2):
                pltpu.async_copy(win.at[pl.ds(kx * M + m0, PIECE)],
                                 flat.at[pl.ds(kx * PIECE, PIECE)], semw)

            @pl.when(s == last)
            def _tgather():
                for kx in range(D2):
                    pltpu.async_copy(win.at[pl.ds(kx * M + MAIN, TAILM)],
                                     tbuf.at[pl.ds(kx * TAILM, TAILM)], semw)

            for kx in range(D2):
                pltpu.make_async_copy(
                    win.at[pl.ds(kx * M + m0, PIECE)],
                    flat.at[pl.ds(kx * PIECE, PIECE)], semw).wait()

            @pl.when(s == last)
            def _tgwait():
                for kx in range(D2):
                    pltpu.make_async_copy(
                        win.at[pl.ds(kx * M + MAIN, TAILM)],
                        tbuf.at[pl.ds(kx * TAILM, TAILM)], semw).wait()

            @pl.when(jj < SPC - 1)
            def _rz():
                issue_zero()

            @pl.when(jj > 0)
            def _wo():
                pltpu.make_async_copy(
                    stgo, out_hbm.at[j - 1, :, pl.ds(m0, PIECE)], semo).wait()

            wait_input(j)
            for kx in range(D2):
                @pl.loop(0, PIECE // 16, step=4)
                def _add(v):
                    for u in range(4):
                        o = (v + u) * 16
                        stgo[kx, pl.ds(o, 16)] = (
                            stg[kx, pl.ds(o, 16)]
                            + flat[pl.ds(kx * PIECE + o, 16)])

            @pl.when(s == last)
            def _tout():
                pltpu.sync_copy(tbuf, tout_hbm.at[pl.ds(j * 128, 128)])

            pltpu.async_copy(stgo, out_hbm.at[j, :, pl.ds(m0, PIECE)], semo)

            @pl.when(jj < SPC - 1)
            def _pfi():
                issue_input(j + 1)

        pltpu.make_async_copy(
            stgo, out_hbm.at[j0 + SPC - 1, :, pl.ds(m0, PIECE)], semo).wait()

    return k(in3, idx3, src3)


def kernel(input, index, src):
    out3, tout = _sc_scatter_add(
        jnp.transpose(input, (1, 2, 0)),
        jnp.transpose(index, (1, 2, 0)),
        jnp.transpose(src, (1, 2, 0)),
    )
    out = jnp.transpose(out3, (2, 0, 1))
    tail = jnp.transpose(tout.reshape(D1, D2, TAILM), (2, 0, 1))
    tail = tail + lax.slice(input, (MAIN, 0, 0), (M, D1, D2))
    return lax.dynamic_update_slice(out, tail, (MAIN, 0, 0))


# speedup vs baseline: 70.1606x; 1.0945x over previous
import functools

import jax
import jax.numpy as jnp
from jax import lax
from jax.experimental import pallas as pl
from jax.experimental.pallas import tpu as pltpu
from jax.experimental.pallas import tpu_sc as plsc

M, D1, D2 = 100000, 64, 4
B = 16384

NC, NS = 2, 16
SPC = D1 // NC
SLAB = D2 * M
MAIN = 99968
TAILM = M - MAIN
PIECE = 6272
HPIECE = PIECE // 2
UPT = B // NS
VPT = UPT // 16


def _sc_scatter_add(in3, idx3, src3):
    mesh = plsc.VectorSubcoreMesh(core_axis_name="c", subcore_axis_name="s")

    @functools.partial(
        pl.kernel,
        out_type=(
            jax.ShapeDtypeStruct((D1, D2, M), jnp.float32),
            jax.ShapeDtypeStruct((D1 * 128, ), jnp.float32),
        ),
        mesh=mesh,
        scratch_types=[
            pltpu.VMEM_SHARED((SLAB,), jnp.float32),
            pltpu.VMEM((D2, PIECE), jnp.float32),
            pltpu.VMEM((D2, PIECE), jnp.float32),
            pltpu.VMEM((D2 * PIECE,), jnp.float32),
            pltpu.VMEM((D2, UPT), jnp.int32),
            pltpu.VMEM((D2, UPT), jnp.float32),
            pltpu.VMEM((D2 * UPT,), jnp.int32),
            pltpu.VMEM((D2 * UPT,), jnp.float32),
            pltpu.VMEM((D2 * UPT,), jnp.int32),
            pltpu.VMEM((D2 * UPT,), jnp.float32),
            pltpu.VMEM((128,), jnp.float32),
            pltpu.VMEM((HPIECE,), jnp.float32),
            pltpu.SemaphoreType.DMA,
            pltpu.SemaphoreType.DMA,
            pltpu.SemaphoreType.DMA,
            pltpu.SemaphoreType.DMA,
            pltpu.SemaphoreType.DMA,
            pltpu.SemaphoreType.DMA,
        ],
    )
    def k(in_hbm, idx_hbm, src_hbm, out_hbm, tout_hbm, win, stg, stgo, flat,
          idxv, srcv, dl0, sf0, dl1, sf1, tbuf, zflat, semi, semu, semz,
          semw, semo, sems):
        c = lax.axis_index("c")
        s = lax.axis_index("s")
        m0 = jnp.minimum(s * PIECE, MAIN - PIECE)
        i0 = s * UPT
        j0 = c * SPC
        last = NS - 1

        @pl.loop(0, HPIECE // 16, step=4)
        def _z(v):
            for u in range(4):
                zflat[pl.ds((v + u) * 16, 16)] = jnp.zeros((16,), jnp.float32)

        def issue_updates(j):
            pltpu.async_copy(idx_hbm.at[j, :, pl.ds(i0, UPT)], idxv, semu)
            pltpu.async_copy(src_hbm.at[j, :, pl.ds(i0, UPT)], srcv, semu)

        def wait_updates(j):
            pltpu.make_async_copy(
                idx_hbm.at[j, :, pl.ds(i0, UPT)], idxv, semu).wait()
            pltpu.make_async_copy(
                src_hbm.at[j, :, pl.ds(i0, UPT)], srcv, semu).wait()

        def compute_dests(dl, sf):
            for kx in range(D2):
                @pl.loop(0, VPT, step=4)
                def _vec(v):
                    for u in range(4):
                        o = (v + u) * 16
                        dl[pl.ds(kx * UPT + o, 16)] = (
                            idxv[kx, pl.ds(o, 16)] + kx * M)
                        sf[pl.ds(kx * UPT + o, 16)] = srcv[kx, pl.ds(o, 16)]

        def issue_zero():
            for kx in range(D2):
                for h in range(2):
                    pltpu.async_copy(
                        zflat,
                        win.at[pl.ds(kx * M + m0 + h * HPIECE, HPIECE)], semz)

            @pl.when(s == last)
            def _tz():
                for kx in range(D2):
                    pltpu.async_copy(zflat.at[pl.ds(0, TAILM)],
                                     win.at[pl.ds(kx * M + MAIN, TAILM)],
                                     semz)

        def wait_zero():
            for kx in range(D2):
                for h in range(2):
                    pltpu.make_async_copy(
                        zflat,
                        win.at[pl.ds(kx * M + m0 + h * HPIECE, HPIECE)],
                        semz).wait()

            @pl.when(s == last)
            def _tzw():
                for kx in range(D2):
                    pltpu.make_async_copy(
                        zflat.at[pl.ds(0, TAILM)],
                        win.at[pl.ds(kx * M + MAIN, TAILM)], semz).wait()

        def issue_input(j):
            pltpu.async_copy(in_hbm.at[j, :, pl.ds(m0, PIECE)], stg, semi)

        def wait_input(j):
            pltpu.make_async_copy(
                in_hbm.at[j, :, pl.ds(m0, PIECE)], stg, semi).wait()

        def body(j, jj, dl, sf, dln, sfn):
            pltpu.async_copy(sf, win.at[dl], sems, add=True)

            @pl.when(j >= j0 + 2)
            def _wo():
                pltpu.make_async_copy(
                    stgo, out_hbm.at[j - 2, :, pl.ds(m0, PIECE)], semo).wait()

            @pl.when(j >= j0 + 1)
            def _fin_prev():
                wait_input(j - 1)
                for kx in range(D2):
                    @pl.loop(0, PIECE // 16, step=4)
                    def _add(v):
                        for u in range(4):
                            o = (v + u) * 16
                            stgo[kx, pl.ds(o, 16)] = (
                                stg[kx, pl.ds(o, 16)]
                                + flat[pl.ds(kx * PIECE + o, 16)])

                @pl.when(s == last)
                def _tout():
                    pltpu.sync_copy(tbuf,
                                    tout_hbm.at[pl.ds((j - 1) * 128, 128)])

                pltpu.async_copy(stgo, out_hbm.at[j - 1, :, pl.ds(m0, PIECE)],
                                 semo)
                issue_input(j)

            pltpu.make_async_copy(sf, win.at[dl], sems).wait()
            plsc.subcore_barrier()

            for kx in range(D2):
                pltpu.async_copy(win.at[pl.ds(kx * M + m0, PIECE)],
                                 flat.at[pl.ds(kx * PIECE, PIECE)], semw)

            @pl.when(s == last)
            def _tgather():
                for kx in range(D2):
                    pltpu.async_copy(win.at[pl.ds(kx * M + MAIN, TAILM)],
                                     tbuf.at[pl.ds(kx * TAILM, TAILM)], semw)

            @pl.when(j < j0 + SPC - 1)
            def _next_dests():
                wait_updates(j + 1)
                compute_dests(dln, sfn)

            @pl.when(j < j0 + SPC - 2)
            def _pfu():
                issue_updates(j + 2)

            for kx in range(D2):
                pltpu.make_async_copy(
                    win.at[pl.ds(kx * M + m0, PIECE)],
                    flat.at[pl.ds(kx * PIECE, PIECE)], semw).wait()

            @pl.when(s == last)
            def _tgwait():
                for kx in range(D2):
                    pltpu.make_async_copy(
                        win.at[pl.ds(kx * M + MAIN, TAILM)],
                        tbuf.at[pl.ds(kx * TAILM, TAILM)], semw).wait()

            @pl.when(j < j0 + SPC - 1)
            def _rz():
                issue_zero()
                wait_zero()
                plsc.subcore_barrier()

        issue_updates(j0)
        issue_input(j0)
        issue_zero()
        wait_updates(j0)
        compute_dests(dl0, sf0)
        issue_updates(j0 + 1)
        wait_zero()
        plsc.subcore_barrier()

        @pl.loop(0, SPC, step=2)
        def _slab2(jj):
            body(j0 + jj, jj, dl0, sf0, dl1, sf1)
            body(j0 + jj + 1, jj + 1, dl1, sf1, dl0, sf0)

        jl = j0 + SPC - 1
        pltpu.make_async_copy(
            stgo, out_hbm.at[jl - 1, :, pl.ds(m0, PIECE)], semo).wait()
        wait_input(jl)
        for kx in range(D2):
            @pl.loop(0, PIECE // 16, step=4)
            def _adde(v):
                for u in range(4):
                    o = (v + u) * 16
                    stgo[kx, pl.ds(o, 16)] = (
                        stg[kx, pl.ds(o, 16)]
                        + flat[pl.ds(kx * PIECE + o, 16)])

        @pl.when(s == last)
        def _toute():
            pltpu.sync_copy(tbuf, tout_hbm.at[pl.ds(jl * 128, 128)])

        pltpu.sync_copy(stgo, out_hbm.at[jl, :, pl.ds(m0, PIECE)])

    return k(in3, idx3, src3)


def kernel(input, index, src):
    out3, tout = _sc_scatter_add(
        jnp.transpose(input, (1, 2, 0)),
        jnp.transpose(index, (1, 2, 0)),
        jnp.transpose(src, (1, 2, 0)),
    )
    out = jnp.transpose(out3, (2, 0, 1))
    tail = jnp.transpose(tout.reshape(D1, D2, TAILM), (2, 0, 1))
    tail = tail + lax.slice(input, (MAIN, 0, 0), (M, D1, D2))
    return lax.dynamic_update_slice(out, tail, (MAIN, 0, 0))
